# baseline (speedup 1.0000x reference)
import functools
import jax
import jax.numpy as jnp
from jax import lax
from jax.experimental import pallas as pl
from jax.experimental.pallas import tpu as pltpu
from jax.experimental.pallas import tpu_sc as plsc

B, C, H = 4, 2048, 2048
L = 16

HTC = 1536
HT = 1536


CSPLIT = 2
CT = C // CSPLIT


def _tc_body(x_ref, o_ref, s_ref, w_ref):
    c = pl.program_id(1)
    x = x_ref[0]
    xc = jnp.clip(x, -20.0, 20.0)
    e = jnp.exp(xc - 20.0)
    s = jnp.sum(e, axis=0)
    w = jnp.sum(e * x, axis=0)

    @pl.when(c == 0)
    def _init():
        s_ref[...] = s
        w_ref[...] = w

    @pl.when(c == CSPLIT - 1)
    def _fini():
        o_ref[0, 0] = (w_ref[...] + w) / (s_ref[...] + s)

    @pl.when(jnp.logical_and(c > 0, c < CSPLIT - 1))
    def _acc():
        s_ref[...] = s_ref[...] + s
        w_ref[...] = w_ref[...] + w


def _tc_pool(x):
    return pl.pallas_call(
        _tc_body,
        grid=(B, CSPLIT),
        in_specs=[pl.BlockSpec((1, CT, HT), lambda b, c: (b, c, 0))],
        out_specs=pl.BlockSpec((1, 1, HT), lambda b, c: (b, 0, 0)),
        out_shape=jax.ShapeDtypeStruct((B, 1, HTC), x.dtype),
        scratch_shapes=[
            pltpu.VMEM((HT,), jnp.float32),
            pltpu.VMEM((HT,), jnp.float32),
        ],
    )(x)


HSC = H - HTC
COLS = 128
G = COLS // L
NCB = HSC // COLS
NRS = 32 // NCB
RSEG = C // NRS
CHUNK = 256
NCHUNK = RSEG // CHUNK
NBUF = 2


def _acc_chunk(buf, accs):
    def row(i, accs):
        out = list(accs)
        for g in range(G):
            v = buf[i, pl.ds(g * L, L)]
            xc = jnp.minimum(jnp.maximum(v, -20.0), 20.0)
            e = jnp.exp(xc - 20.0)
            out[g] = out[g] + e
            out[G + g] = out[G + g] + e * v
        return tuple(out)

    return plsc.parallel_loop(0, CHUNK, step=1, unroll=2, carry=tuple(accs))(row)


@functools.partial(
    pl.kernel,
    mesh=plsc.VectorSubcoreMesh(core_axis_name="c", subcore_axis_name="s"),
    out_type=jax.ShapeDtypeStruct((B * HSC,), jnp.float32),
    scratch_types=[
        pltpu.VMEM((NBUF, CHUNK, COLS), jnp.float32),
        pltpu.VMEM((B, 2 * COLS), jnp.float32),
        pltpu.VMEM((NRS, B, 2 * COLS), jnp.float32),
        pltpu.VMEM((COLS,), jnp.float32),
        pltpu.VMEM_SHARED((16, B, 2 * COLS), jnp.float32),
        [pltpu.SemaphoreType.DMA] * NBUF,
        pltpu.SemaphoreType.DMA,
    ],
)
def _sc_pool(x_hbm, out_hbm, buf, part, comb, obuf, shared, sems, csem):
    cid = lax.axis_index("c")
    sid = lax.axis_index("s")
    cb = cid * 2 + sid // NRS
    rs = sid % NRS
    col0 = HTC + cb * COLS
    row0 = rs * RSEG

    sched = [(b, ch) for b in range(B) for ch in range(NCHUNK)]
    total = len(sched)

    def start(k):
        b, ch = sched[k]
        return pltpu.async_copy(
            x_hbm.at[pl.ds(b * C + row0 + ch * CHUNK, CHUNK), pl.ds(col0, COLS)],
            buf.at[k % NBUF],
            sems[k % NBUF],
        )

    copies = [None] * total
    for k in range(NBUF - 1):
        copies[k] = start(k)

    accs = tuple(jnp.zeros((L,), jnp.float32) for _ in range(2 * G))
    for k in range(total):
        if k + NBUF - 1 < total:
            copies[k + NBUF - 1] = start(k + NBUF - 1)
        copies[k].wait()
        accs = _acc_chunk(buf.at[k % NBUF], accs)
        b, ch = sched[k]
        if ch == NCHUNK - 1:
            for g in range(G):
                part[b, pl.ds(g * L, L)] = accs[g]
                part[b, pl.ds(COLS + g * L, L)] = accs[G + g]
            accs = tuple(jnp.zeros((L,), jnp.float32) for _ in range(2 * G))

    pltpu.sync_copy(part, shared.at[sid])
    plsc.subcore_barrier()

    @pl.when(rs == 0)
    def _combine():
        pltpu.sync_copy(shared.at[pl.ds(sid, NRS)], comb)
        for b in range(B):
            for g in range(G):
                s = comb[0, b, pl.ds(g * L, L)]
                w = comb[0, b, pl.ds(COLS + g * L, L)]
                for i in range(1, NRS):
                    s = s + comb[i, b, pl.ds(g * L, L)]
                    w = w + comb[i, b, pl.ds(COLS + g * L, L)]
                obuf[pl.ds(g * L, L)] = w / s
            pltpu.async_copy(
                obuf, out_hbm.at[pl.ds(b * HSC + cb * COLS, COLS)], csem
            ).wait()


def kernel(x):
    assert x.shape == (B, C, H)
    sc = _sc_pool(x.reshape(B * C, H))
    tc = _tc_pool(x)
    return jnp.concatenate([tc.reshape(B, HTC), sc.reshape(B, HSC)], axis=1)


# speedup vs baseline: 1.0292x; 1.0292x over previous
import functools
import jax
import jax.numpy as jnp
from jax import lax
from jax.experimental import pallas as pl
from jax.experimental.pallas import tpu as pltpu
from jax.experimental.pallas import tpu_sc as plsc

B, C, H = 4, 2048, 2048
L = 16

HTC = 1536
HT = 1536


def _tc_body(x_ref, o_ref):
    x = x_ref[0]
    xc = jnp.clip(x, -20.0, 20.0)
    e = jnp.exp(xc - 20.0)
    s = jnp.sum(e, axis=0)
    w = jnp.sum(e * x, axis=0)
    o_ref[0, 0] = w / s


def _tc_pool(x):
    return pl.pallas_call(
        _tc_body,
        grid=(B, HTC // HT),
        in_specs=[pl.BlockSpec((1, C, HT), lambda b, h: (b, 0, h))],
        out_specs=pl.BlockSpec((1, 1, HT), lambda b, h: (b, 0, h)),
        out_shape=jax.ShapeDtypeStruct((B, 1, HTC), x.dtype),
    )(x)


HSC = H - HTC
COLS = 128
G = COLS // L
NCB = HSC // COLS
NRS = 32 // NCB
RSEG = C // NRS
CHUNK = 256
NCHUNK = RSEG // CHUNK
NBUF = 2


def _acc_chunk(buf, accs):
    def row(i, accs):
        out = list(accs)
        for g in range(G):
            v = buf[i, pl.ds(g * L, L)]
            xc = jnp.minimum(jnp.maximum(v, -20.0), 20.0)
            e = jnp.exp(xc - 20.0)
            out[g] = out[g] + e
            out[G + g] = out[G + g] + e * v
        return tuple(out)

    return plsc.parallel_loop(0, CHUNK, step=1, unroll=2, carry=tuple(accs))(row)


@functools.partial(
    pl.kernel,
    mesh=plsc.VectorSubcoreMesh(core_axis_name="c", subcore_axis_name="s"),
    out_type=jax.ShapeDtypeStruct((B * HSC,), jnp.float32),
    scratch_types=[
        pltpu.VMEM((NBUF, CHUNK, COLS), jnp.float32),
        pltpu.VMEM((B, 2 * COLS), jnp.float32),
        pltpu.VMEM((NRS, B, 2 * COLS), jnp.float32),
        pltpu.VMEM((COLS,), jnp.float32),
        pltpu.VMEM_SHARED((16, B, 2 * COLS), jnp.float32),
        [pltpu.SemaphoreType.DMA] * NBUF,
        pltpu.SemaphoreType.DMA,
    ],
)
def _sc_pool(x_hbm, out_hbm, buf, part, comb, obuf, shared, sems, csem):
    cid = lax.axis_index("c")
    sid = lax.axis_index("s")
    cb = cid * 2 + sid // NRS
    rs = sid % NRS
    col0 = HTC + cb * COLS
    row0 = rs * RSEG

    sched = [(b, ch) for b in range(B) for ch in range(NCHUNK)]
    total = len(sched)

    def start(k):
        b, ch = sched[k]
        return pltpu.async_copy(
            x_hbm.at[pl.ds(b * C + row0 + ch * CHUNK, CHUNK), pl.ds(col0, COLS)],
            buf.at[k % NBUF],
            sems[k % NBUF],
        )

    copies = [None] * total
    for k in range(NBUF - 1):
        copies[k] = start(k)

    accs = tuple(jnp.zeros((L,), jnp.float32) for _ in range(2 * G))
    for k in range(total):
        if k + NBUF - 1 < total:
            copies[k + NBUF - 1] = start(k + NBUF - 1)
        copies[k].wait()
        accs = _acc_chunk(buf.at[k % NBUF], accs)
        b, ch = sched[k]
        if ch == NCHUNK - 1:
            for g in range(G):
                part[b, pl.ds(g * L, L)] = accs[g]
                part[b, pl.ds(COLS + g * L, L)] = accs[G + g]
            accs = tuple(jnp.zeros((L,), jnp.float32) for _ in range(2 * G))

    pltpu.sync_copy(part, shared.at[sid])
    plsc.subcore_barrier()

    @pl.when(rs == 0)
    def _combine():
        pltpu.sync_copy(shared.at[pl.ds(sid, NRS)], comb)
        for b in range(B):
            for g in range(G):
                s = comb[0, b, pl.ds(g * L, L)]
                w = comb[0, b, pl.ds(COLS + g * L, L)]
                for i in range(1, NRS):
                    s = s + comb[i, b, pl.ds(g * L, L)]
                    w = w + comb[i, b, pl.ds(COLS + g * L, L)]
                obuf[pl.ds(g * L, L)] = w / s
            pltpu.async_copy(
                obuf, out_hbm.at[pl.ds(b * HSC + cb * COLS, COLS)], csem
            ).wait()


def kernel(x):
    assert x.shape == (B, C, H)
    sc = _sc_pool(x.reshape(B * C, H))
    tc = _tc_pool(x)
    return jnp.concatenate([tc.reshape(B, HTC), sc.reshape(B, HSC)], axis=1)
